# Initial kernel scaffold; baseline (speedup 1.0000x reference)
import jax
import jax.numpy as jnp
from jax.experimental import pallas as pl


def kernel(x_s, edge_attr_s, W_node, b_node, g_node, beta_node, emb0, emb1, emb2, emb3, W_edge, b_edge, g_edge, beta_edge):
    raise NotImplementedError("write your pallas kernel here")


# speedup vs baseline: 3.5239x; 3.5239x over previous
import functools

import numpy as np
import jax
import jax.numpy as jnp
from jax import lax
from jax.experimental import pallas as pl
from jax.experimental.pallas import tpu as pltpu
from jax.experimental.pallas import tpu_sc as plsc

HIDDEN = 64
NUM_ATTR = 4
VOCAB = 300
VPAD = 304
E = 320000
N = 10000
D_NODE = 128

def _pe_const():
    p = np.arange(VOCAB, dtype=np.float64)[:, None]
    i = np.arange(0, HIDDEN, 2, dtype=np.float64)
    div = np.exp(-(i * (np.log(10000.0) / HIDDEN)))
    ang = p * div[None, :]
    pe = np.stack([np.sin(ang), np.cos(ang)], axis=-1).reshape(VOCAB, HIDDEN)
    out = np.zeros((VPAD, HIDDEN), dtype=np.float32)
    out[:VOCAB] = pe.astype(np.float32)
    return out

_PE_PAD = _pe_const()


def _tables_body(pe, e0, e1, e2, e3, w, b, out):
    w0 = w[0:64, :]
    w1 = w[64:128, :]
    w2 = w[128:192, :]
    w3 = w[192:256, :]
    ws = w0 + w1 + w2 + w3
    out[0] = jnp.dot(pe[...], ws, preferred_element_type=jnp.float32) + b[...]
    out[1] = jnp.dot(e0[...], w0, preferred_element_type=jnp.float32)
    out[2] = jnp.dot(e1[...], w1, preferred_element_type=jnp.float32)
    out[3] = jnp.dot(e2[...], w2, preferred_element_type=jnp.float32)
    out[4] = jnp.dot(e3[...], w3, preferred_element_type=jnp.float32)


def _build_tables(pe, e0, e1, e2, e3, w_edge, b_edge):
    out = pl.pallas_call(
        _tables_body,
        out_shape=jax.ShapeDtypeStruct((5, VPAD, HIDDEN), jnp.float32),
    )(pe, e0, e1, e2, e3, w_edge, b_edge)
    return out.reshape(5 * VPAD, HIDDEN)


NW = 32
EPT = E // NW
CHUNK = 80
NCHUNK = EPT // CHUNK


def _edge_gather_sum(tall, attr_flat):
    mesh = plsc.VectorSubcoreMesh(core_axis_name="c", subcore_axis_name="s")

    @functools.partial(
        pl.kernel,
        out_type=jax.ShapeDtypeStruct((E, HIDDEN), jnp.float32),
        mesh=mesh,
        compiler_params=pltpu.CompilerParams(use_tc_tiling_on_sc=False),
        scratch_types=[
            pltpu.VMEM((5 * CHUNK,), jnp.int32),
            pltpu.VMEM((5, CHUNK), jnp.int32),
            pltpu.VMEM((5 * CHUNK, HIDDEN), jnp.float32),
            pltpu.VMEM((CHUNK, HIDDEN), jnp.float32),
            pltpu.SemaphoreType.DMA,
        ],
    )
    def body(tall_hbm, attr_hbm, y_hbm, attr_v, idx_v, rows5, out_v, sem):
        wid = lax.axis_index("s") * 2 + lax.axis_index("c")
        lane = lax.iota(jnp.int32, 16)

        def chunk(ch, carry):
            base = wid * EPT + ch * CHUNK
            pltpu.sync_copy(attr_hbm.at[pl.ds(base * 5, 5 * CHUNK)], attr_v)
            for j in range(5 * CHUNK // 16):
                v = attr_v[pl.ds(j * 16, 16)]
                off = ((lane + j * 16) % 5) * VPAD
                idx_v[j // 5, pl.ds((j % 5) * 16, 16)] = v + off
            copies = [
                pltpu.async_copy(tall_hbm.at[idx_v.at[q]],
                                 rows5.at[pl.ds(q * CHUNK, CHUNK)], sem)
                for q in range(5)
            ]
            for cp in copies:
                cp.wait()

            def esum(e, c2):
                for cc in range(HIDDEN // 16):
                    sl = pl.ds(cc * 16, 16)
                    out_v[e, sl] = (rows5[5 * e, sl] + rows5[5 * e + 1, sl]
                                    + rows5[5 * e + 2, sl]
                                    + rows5[5 * e + 3, sl]
                                    + rows5[5 * e + 4, sl])
                return c2

            lax.fori_loop(0, CHUNK, esum, 0, unroll=4)
            pltpu.sync_copy(out_v, y_hbm.at[pl.ds(base, CHUNK)])
            return carry

        lax.fori_loop(0, NCHUNK, chunk, 0)

    return body(tall, attr_flat)


def _ln_body(y, g, b, out):
    h = y[...]
    mu = jnp.mean(h, axis=-1, keepdims=True)
    var = jnp.mean((h - mu) * (h - mu), axis=-1, keepdims=True)
    out[...] = (h - mu) * lax.rsqrt(var + 1e-5) * g[...] + b[...]


def _edge_ln(y, g, b):
    blk = 2000
    return pl.pallas_call(
        _ln_body,
        grid=(E // blk,),
        in_specs=[
            pl.BlockSpec((blk, HIDDEN), lambda i: (i, 0)),
            pl.BlockSpec((HIDDEN,), lambda i: (0,)),
            pl.BlockSpec((HIDDEN,), lambda i: (0,)),
        ],
        out_specs=pl.BlockSpec((blk, HIDDEN), lambda i: (i, 0)),
        out_shape=jax.ShapeDtypeStruct((E, HIDDEN), jnp.float32),
    )(y, g, b)


def _node_body(x, w, b, g, be, out):
    h = jnp.dot(x[...], w[...], preferred_element_type=jnp.float32) + b[...]
    mu = jnp.mean(h, axis=-1, keepdims=True)
    var = jnp.mean((h - mu) * (h - mu), axis=-1, keepdims=True)
    out[...] = (h - mu) * lax.rsqrt(var + 1e-5) * g[...] + be[...]


def _node_transform(x, w, b, g, be):
    blk = 1000
    return pl.pallas_call(
        _node_body,
        grid=(N // blk,),
        in_specs=[
            pl.BlockSpec((blk, D_NODE), lambda i: (i, 0)),
            pl.BlockSpec((D_NODE, HIDDEN), lambda i: (0, 0)),
            pl.BlockSpec((HIDDEN,), lambda i: (0,)),
            pl.BlockSpec((HIDDEN,), lambda i: (0,)),
            pl.BlockSpec((HIDDEN,), lambda i: (0,)),
        ],
        out_specs=pl.BlockSpec((blk, HIDDEN), lambda i: (i, 0)),
        out_shape=jax.ShapeDtypeStruct((N, HIDDEN), jnp.float32),
    )(x, w, b, g, be)


def kernel(x_s, edge_attr_s, W_node, b_node, g_node, beta_node,
           emb0, emb1, emb2, emb3, W_edge, b_edge, g_edge, beta_edge):
    pe = jnp.asarray(_PE_PAD)
    pad = ((0, VPAD - VOCAB), (0, 0))
    tall = _build_tables(pe,
                         jnp.pad(emb0, pad), jnp.pad(emb1, pad),
                         jnp.pad(emb2, pad), jnp.pad(emb3, pad),
                         W_edge, b_edge)
    attr_flat = edge_attr_s.reshape(-1)
    y = _edge_gather_sum(tall, attr_flat)
    h_e = _edge_ln(y, g_edge, beta_edge)
    h_x = _node_transform(x_s, W_node, b_node, g_node, beta_node)
    return (h_x, h_e)


# speedup vs baseline: 4.5673x; 1.2961x over previous
import functools

import numpy as np
import jax
import jax.numpy as jnp
from jax import lax
from jax.experimental import pallas as pl
from jax.experimental.pallas import tpu as pltpu
from jax.experimental.pallas import tpu_sc as plsc

HIDDEN = 64
NUM_ATTR = 4
VOCAB = 300
VPAD = 304
E = 320000
N = 10000
D_NODE = 128

def _pe_const():
    p = np.arange(VOCAB, dtype=np.float64)[:, None]
    i = np.arange(0, HIDDEN, 2, dtype=np.float64)
    div = np.exp(-(i * (np.log(10000.0) / HIDDEN)))
    ang = p * div[None, :]
    pe = np.stack([np.sin(ang), np.cos(ang)], axis=-1).reshape(VOCAB, HIDDEN)
    out = np.zeros((VPAD, HIDDEN), dtype=np.float32)
    out[:VOCAB] = pe.astype(np.float32)
    return out

_PE_PAD = _pe_const()


def _tables_body(pe, e0, e1, e2, e3, w, b, out):
    w0 = w[0:64, :]
    w1 = w[64:128, :]
    w2 = w[128:192, :]
    w3 = w[192:256, :]
    ws = w0 + w1 + w2 + w3
    out[0] = jnp.dot(pe[...], ws, preferred_element_type=jnp.float32) + b[...]
    out[1] = jnp.dot(e0[...], w0, preferred_element_type=jnp.float32)
    out[2] = jnp.dot(e1[...], w1, preferred_element_type=jnp.float32)
    out[3] = jnp.dot(e2[...], w2, preferred_element_type=jnp.float32)
    out[4] = jnp.dot(e3[...], w3, preferred_element_type=jnp.float32)


def _build_tables(pe, e0, e1, e2, e3, w_edge, b_edge):
    out = pl.pallas_call(
        _tables_body,
        out_shape=jax.ShapeDtypeStruct((5, VPAD, HIDDEN), jnp.float32),
    )(pe, e0, e1, e2, e3, w_edge, b_edge)
    return out.reshape(5 * VPAD, HIDDEN)


NW = 32
EPT = E // NW
CHUNK = 80
NCHUNK = EPT // CHUNK


def _edge_gather_sum(tall, attr_flat):
    mesh = plsc.VectorSubcoreMesh(core_axis_name="c", subcore_axis_name="s")

    @functools.partial(
        pl.kernel,
        out_type=jax.ShapeDtypeStruct((E, HIDDEN), jnp.float32),
        mesh=mesh,
        compiler_params=pltpu.CompilerParams(use_tc_tiling_on_sc=False),
        scratch_types=[
            pltpu.VMEM((5 * EPT,), jnp.int32),
            pltpu.VMEM((2, 5, CHUNK), jnp.int32),
            pltpu.VMEM((2, 5 * CHUNK, HIDDEN), jnp.float32),
            pltpu.VMEM((2, CHUNK, HIDDEN), jnp.float32),
            pltpu.SemaphoreType.DMA,
            pltpu.SemaphoreType.DMA,
            pltpu.SemaphoreType.DMA,
            pltpu.SemaphoreType.DMA,
        ],
    )
    def body(tall_hbm, attr_hbm, y_hbm, attr_v, idx_v, rows5, out_v,
             gsem0, gsem1, osem0, osem1):
        wid = lax.axis_index("s") * 2 + lax.axis_index("c")
        lane = lax.iota(jnp.int32, 16)
        gsems = (gsem0, gsem1)
        osems = (osem0, osem1)

        pltpu.sync_copy(attr_hbm.at[pl.ds(wid * (5 * EPT), 5 * EPT)], attr_v)

        def build_idx(ch, buf):
            for j in range(5 * CHUNK // 16):
                v = attr_v[pl.ds(ch * (5 * CHUNK) + j * 16, 16)]
                off = ((lane + j * 16) % 5) * VPAD
                idx_v[buf, j // 5, pl.ds((j % 5) * 16, 16)] = v + off

        def fire_gathers(buf):
            for q in range(5):
                pltpu.async_copy(tall_hbm.at[idx_v.at[buf, q]],
                                 rows5.at[buf, pl.ds(q * CHUNK, CHUNK)],
                                 gsems[buf])

        def drain_gathers(buf):
            for q in range(5):
                pltpu.make_async_copy(
                    tall_hbm.at[idx_v.at[buf, q]],
                    rows5.at[buf, pl.ds(q * CHUNK, CHUNK)],
                    gsems[buf]).wait()

        build_idx(0, 0)
        fire_gathers(0)

        def process(ch, buf, prefetch, reclaim):
            if prefetch:
                build_idx(ch + 1, 1 - buf)
                fire_gathers(1 - buf)
            drain_gathers(buf)

            def _reclaim():
                pltpu.make_async_copy(
                    out_v.at[buf],
                    y_hbm.at[pl.ds(wid * EPT, CHUNK)],
                    osems[buf]).wait()

            if reclaim is None:
                pl.when(ch >= 2)(_reclaim)
            elif reclaim:
                _reclaim()

            def esum(e, c2):
                for cc in range(HIDDEN // 16):
                    sl = pl.ds(cc * 16, 16)
                    out_v[buf, e, sl] = (rows5[buf, 5 * e, sl]
                                         + rows5[buf, 5 * e + 1, sl]
                                         + rows5[buf, 5 * e + 2, sl]
                                         + rows5[buf, 5 * e + 3, sl]
                                         + rows5[buf, 5 * e + 4, sl])
                return c2

            lax.fori_loop(0, CHUNK, esum, 0, unroll=4)
            pltpu.async_copy(out_v.at[buf],
                             y_hbm.at[pl.ds(wid * EPT + ch * CHUNK, CHUNK)],
                             osems[buf])

        def pair(step, carry):
            process(2 * step, 0, True, None)
            process(2 * step + 1, 1, True, None)
            return carry

        lax.fori_loop(0, (NCHUNK - 1) // 2, pair, 0)
        process(NCHUNK - 1, 0, False, True)
        for buf in range(2):
            pltpu.make_async_copy(
                out_v.at[buf],
                y_hbm.at[pl.ds(wid * EPT, CHUNK)],
                osems[buf]).wait()

    return body(tall, attr_flat)


def _ln_body(y, g, b, out):
    h = y[...]
    mu = jnp.mean(h, axis=-1, keepdims=True)
    var = jnp.mean((h - mu) * (h - mu), axis=-1, keepdims=True)
    out[...] = (h - mu) * lax.rsqrt(var + 1e-5) * g[...] + b[...]


def _edge_ln(y, g, b):
    blk = 2000
    return pl.pallas_call(
        _ln_body,
        grid=(E // blk,),
        in_specs=[
            pl.BlockSpec((blk, HIDDEN), lambda i: (i, 0)),
            pl.BlockSpec((HIDDEN,), lambda i: (0,)),
            pl.BlockSpec((HIDDEN,), lambda i: (0,)),
        ],
        out_specs=pl.BlockSpec((blk, HIDDEN), lambda i: (i, 0)),
        out_shape=jax.ShapeDtypeStruct((E, HIDDEN), jnp.float32),
    )(y, g, b)


def _node_body(x, w, b, g, be, out):
    h = jnp.dot(x[...], w[...], preferred_element_type=jnp.float32) + b[...]
    mu = jnp.mean(h, axis=-1, keepdims=True)
    var = jnp.mean((h - mu) * (h - mu), axis=-1, keepdims=True)
    out[...] = (h - mu) * lax.rsqrt(var + 1e-5) * g[...] + be[...]


def _node_transform(x, w, b, g, be):
    blk = 1000
    return pl.pallas_call(
        _node_body,
        grid=(N // blk,),
        in_specs=[
            pl.BlockSpec((blk, D_NODE), lambda i: (i, 0)),
            pl.BlockSpec((D_NODE, HIDDEN), lambda i: (0, 0)),
            pl.BlockSpec((HIDDEN,), lambda i: (0,)),
            pl.BlockSpec((HIDDEN,), lambda i: (0,)),
            pl.BlockSpec((HIDDEN,), lambda i: (0,)),
        ],
        out_specs=pl.BlockSpec((blk, HIDDEN), lambda i: (i, 0)),
        out_shape=jax.ShapeDtypeStruct((N, HIDDEN), jnp.float32),
    )(x, w, b, g, be)


def kernel(x_s, edge_attr_s, W_node, b_node, g_node, beta_node,
           emb0, emb1, emb2, emb3, W_edge, b_edge, g_edge, beta_edge):
    pe = jnp.asarray(_PE_PAD)
    pad = ((0, VPAD - VOCAB), (0, 0))
    tall = _build_tables(pe,
                         jnp.pad(emb0, pad), jnp.pad(emb1, pad),
                         jnp.pad(emb2, pad), jnp.pad(emb3, pad),
                         W_edge, b_edge)
    attr_flat = edge_attr_s.reshape(-1)
    y = _edge_gather_sum(tall, attr_flat)
    h_e = _edge_ln(y, g_edge, beta_edge)
    h_x = _node_transform(x_s, W_node, b_node, g_node, beta_node)
    return (h_x, h_e)
